# Initial kernel scaffold; baseline (speedup 1.0000x reference)
import jax
import jax.numpy as jnp
from jax.experimental import pallas as pl


def kernel(i, sigma):
    raise NotImplementedError("write your pallas kernel here")


# speedup vs baseline: 1.7234x; 1.7234x over previous
import jax
import jax.numpy as jnp
from jax import lax
from jax.experimental import pallas as pl

DIM_ = 100000
ROWS = 256
COLB = 4096


def _fill_body(i_ref, sigma_ref, out_ref):
    j = pl.program_id(0)
    col0 = j * COLB
    move = (1.0 - jnp.exp(-sigma_ref[...])) * (1.0 / DIM_)
    diag = 1.0 - move * float(DIM_ - 1)
    cols = col0 + lax.broadcasted_iota(jnp.int32, (ROWS, COLB), 1)
    out_ref[...] = jnp.where(cols == i_ref[...], diag, move)


def kernel(i, sigma):
    i2 = i.reshape(ROWS, 1)
    s2 = sigma.reshape(ROWS, 1)
    grid = (pl.cdiv(DIM_, COLB),)
    out = pl.pallas_call(
        _fill_body,
        grid=grid,
        in_specs=[
            pl.BlockSpec((ROWS, 1), lambda j: (0, 0)),
            pl.BlockSpec((ROWS, 1), lambda j: (0, 0)),
        ],
        out_specs=pl.BlockSpec((ROWS, COLB), lambda j: (0, j)),
        out_shape=jax.ShapeDtypeStruct((ROWS, DIM_), jnp.float32),
    )(i2, s2)
    return out.reshape(i.shape + (DIM_,))
